# baseline (speedup 1.0000x reference)
import functools

import jax
import jax.numpy as jnp
from jax import lax
from jax.experimental import pallas as pl
from jax.experimental.pallas import tpu as pltpu
from jax.experimental.pallas import tpu_sc as plsc

EMB_DIM = 32
NUM_CODES = 8192
N_TOK = 4096
BETA = 0.25

K_BLK = 1024
J = NUM_CODES // K_BLK

NC, NS = 2, 16
NW = NC * NS
B_PER_W = N_TOK // NW


def _argmin_body(et_ref, xt_ref, idx_ref, loss_ref,
                 rm_ref, ri_ref, xx_ref, rowf_ref, w_ref, bi_ref):
    j = pl.program_id(0)
    et = et_ref[...]
    xt = xt_ref[...]

    @pl.when(j == 0)
    def _():
        xx_ref[...] = jnp.sum(xt * xt, axis=0, keepdims=True)
        rowf_ref[...] = lax.broadcasted_iota(
            jnp.int32, rowf_ref.shape, 0).astype(jnp.float32)
        lane = lax.broadcasted_iota(
            jnp.int32, w_ref.shape, 1).astype(jnp.float32)
        sub = lax.broadcasted_iota(jnp.int32, w_ref.shape, 0)
        w_ref[...] = jnp.where(sub == 0, jnp.float32(1.0), lane)

    e2 = et * (-2.0)
    ee = jnp.sum(et * et, axis=1, keepdims=True)
    s2 = jnp.dot(e2, xt)
    d = (ee + xx_ref[...]) + s2
    bm = jnp.min(d, axis=0, keepdims=True)
    mask = jnp.where(d == bm, jnp.float32(1.0), jnp.float32(0.0))
    cs = lax.dot_general(w_ref[...], mask, (((1,), (0,)), ((), ())),
                         precision=lax.Precision.HIGHEST)
    cnt = cs[0:1, :]
    off = jnp.float32(K_BLK) * j
    bi_ref[...] = cs[1:2, :] + off

    @pl.when(jnp.any(cnt > 1.5))
    def _():
        exact = jnp.min(
            jnp.where(d == bm, rowf_ref[...], jnp.float32(1e9)),
            axis=0, keepdims=True) + off
        bi_ref[...] = jnp.where(cnt > 1.5, exact, bi_ref[...])

    bi = bi_ref[...]

    @pl.when(j == 0)
    def _():
        rm_ref[...] = bm
        ri_ref[...] = bi

    @pl.when(j > 0)
    def _():
        better = bm < rm_ref[...]
        ri_ref[...] = jnp.where(better, bi, ri_ref[...])
        rm_ref[...] = jnp.minimum(bm, rm_ref[...])

    @pl.when(j == J - 1)
    def _():
        idx_ref[...] = ri_ref[...].astype(jnp.int32)
        loss_ref[0, 0] = jnp.sum(rm_ref[...]) * (
            (1.0 + BETA) / (N_TOK * EMB_DIM))


_distance_argmin = pl.pallas_call(
    _argmin_body,
    grid=(J,),
    in_specs=[
        pl.BlockSpec((K_BLK, EMB_DIM), lambda j: (j, 0)),
        pl.BlockSpec((EMB_DIM, N_TOK), lambda j: (0, 0)),
    ],
    out_specs=[
        pl.BlockSpec((1, N_TOK), lambda j: (0, 0)),
        pl.BlockSpec((1, 1), lambda j: (0, 0), memory_space=pltpu.SMEM),
    ],
    out_shape=[
        jax.ShapeDtypeStruct((1, N_TOK), jnp.int32),
        jax.ShapeDtypeStruct((1, 1), jnp.float32),
    ],
    scratch_shapes=[
        pltpu.VMEM((1, N_TOK), jnp.float32),
        pltpu.VMEM((1, N_TOK), jnp.float32),
        pltpu.VMEM((1, N_TOK), jnp.float32),
        pltpu.VMEM((K_BLK, N_TOK), jnp.float32),
        pltpu.VMEM((2, K_BLK), jnp.float32),
        pltpu.VMEM((1, N_TOK), jnp.float32),
    ],
    compiler_params=pltpu.CompilerParams(
        dimension_semantics=("arbitrary",)),
)


@functools.partial(
    pl.kernel,
    mesh=plsc.VectorSubcoreMesh(core_axis_name="c", subcore_axis_name="s"),
    out_type=jax.ShapeDtypeStruct((N_TOK, EMB_DIM), jnp.float32),
    scratch_types=[
        pltpu.VMEM((B_PER_W,), jnp.int32),
        pltpu.VMEM((B_PER_W, EMB_DIM), jnp.float32),
        pltpu.SemaphoreType.DMA,
    ],
    compiler_params=pltpu.CompilerParams(use_tc_tiling_on_sc=False),
)
def _gather_rows(table_hbm, idx_hbm, out_hbm, idx_v, rows_v, sem):
    wid = lax.axis_index("s") * NC + lax.axis_index("c")
    base = wid * B_PER_W
    pltpu.sync_copy(idx_hbm.at[pl.ds(base, B_PER_W)], idx_v)
    pltpu.async_copy(table_hbm.at[idx_v], rows_v, sem).wait()
    pltpu.sync_copy(rows_v, out_hbm.at[pl.ds(base, B_PER_W)])


def kernel(x, embeddings):
    xt = jnp.reshape(x, (-1, EMB_DIM)).T
    table = embeddings.T
    idx2d, loss11 = _distance_argmin(table, xt)
    q = _gather_rows(table, jnp.reshape(idx2d, (-1,)))
    quantized = jnp.reshape(q, x.shape)
    return quantized, loss11[0, 0]


# speedup vs baseline: 1.9229x; 1.9229x over previous
import functools

import jax
import jax.numpy as jnp
from jax import lax
from jax.experimental import pallas as pl
from jax.experimental.pallas import tpu as pltpu
from jax.experimental.pallas import tpu_sc as plsc

EMB_DIM = 32
NUM_CODES = 8192
N_TOK = 4096
BETA = 0.25

K_BLK = 1024
J = NUM_CODES // K_BLK

NC, NS = 2, 16
NW = NC * NS
B_PER_W = N_TOK // NW


def _argmin_body(et_ref, xt_ref, idx_ref, loss_ref,
                 rm_ref, ri_ref, xx_ref, rowf_ref):
    j = pl.program_id(0)
    et = et_ref[...]
    xt = xt_ref[...]

    @pl.when(j == 0)
    def _():
        xx_ref[...] = jnp.sum(xt * xt, axis=0, keepdims=True)
        rowf_ref[...] = lax.broadcasted_iota(
            jnp.int32, rowf_ref.shape, 0).astype(jnp.float32)

    e2 = et * (-2.0)
    ee = jnp.sum(et * et, axis=1, keepdims=True)
    s2 = jnp.dot(e2, xt)
    d = (ee + xx_ref[...]) + s2
    bm = jnp.min(d, axis=0, keepdims=True)
    bi = jnp.min(jnp.where(d == bm, rowf_ref[:, 0:1], jnp.float32(1e9)),
                 axis=0, keepdims=True) + jnp.float32(K_BLK) * j

    @pl.when(j == 0)
    def _():
        rm_ref[...] = bm
        ri_ref[...] = bi

    @pl.when(j > 0)
    def _():
        better = bm < rm_ref[...]
        ri_ref[...] = jnp.where(better, bi, ri_ref[...])
        rm_ref[...] = jnp.minimum(bm, rm_ref[...])

    @pl.when(j == J - 1)
    def _():
        idx_ref[...] = ri_ref[...].astype(jnp.int32)
        loss_ref[0, 0] = jnp.sum(rm_ref[...]) * (
            (1.0 + BETA) / (N_TOK * EMB_DIM))


_distance_argmin = pl.pallas_call(
    _argmin_body,
    grid=(J,),
    in_specs=[
        pl.BlockSpec((K_BLK, EMB_DIM), lambda j: (j, 0)),
        pl.BlockSpec((EMB_DIM, N_TOK), lambda j: (0, 0)),
    ],
    out_specs=[
        pl.BlockSpec((1, N_TOK), lambda j: (0, 0)),
        pl.BlockSpec((1, 1), lambda j: (0, 0), memory_space=pltpu.SMEM),
    ],
    out_shape=[
        jax.ShapeDtypeStruct((1, N_TOK), jnp.int32),
        jax.ShapeDtypeStruct((1, 1), jnp.float32),
    ],
    scratch_shapes=[
        pltpu.VMEM((1, N_TOK), jnp.float32),
        pltpu.VMEM((1, N_TOK), jnp.float32),
        pltpu.VMEM((1, N_TOK), jnp.float32),
        pltpu.VMEM((K_BLK, 128), jnp.float32),
    ],
    compiler_params=pltpu.CompilerParams(
        dimension_semantics=("arbitrary",)),
)


@functools.partial(
    pl.kernel,
    mesh=plsc.VectorSubcoreMesh(core_axis_name="c", subcore_axis_name="s"),
    out_type=jax.ShapeDtypeStruct((N_TOK, EMB_DIM), jnp.float32),
    scratch_types=[
        pltpu.VMEM((B_PER_W,), jnp.int32),
        pltpu.VMEM((B_PER_W, EMB_DIM), jnp.float32),
        pltpu.SemaphoreType.DMA,
    ],
    compiler_params=pltpu.CompilerParams(use_tc_tiling_on_sc=False),
)
def _gather_rows(table_hbm, idx_hbm, out_hbm, idx_v, rows_v, sem):
    wid = lax.axis_index("s") * NC + lax.axis_index("c")
    base = wid * B_PER_W
    pltpu.sync_copy(idx_hbm.at[pl.ds(base, B_PER_W)], idx_v)
    pltpu.async_copy(table_hbm.at[idx_v], rows_v, sem).wait()
    pltpu.sync_copy(rows_v, out_hbm.at[pl.ds(base, B_PER_W)])


def kernel(x, embeddings):
    xt = jnp.reshape(x, (-1, EMB_DIM)).T
    table = embeddings.T
    idx2d, loss11 = _distance_argmin(table, xt)
    q = _gather_rows(table, jnp.reshape(idx2d, (-1,)))
    quantized = jnp.reshape(q, x.shape)
    return quantized, loss11[0, 0]
